# baseline (speedup 1.0000x reference)
import functools

import jax
import jax.numpy as jnp
from jax import lax
from jax.experimental import pallas as pl
from jax.experimental.pallas import tpu as pltpu
from jax.experimental.pallas import tpu_sc as plsc

B = 16384
A = 200
NC = 2
NS = 16
NW = NC * NS
R = B // NW
L = 16


def _sc_body(species_hbm, energies_hbm, table_hbm, out_hbm,
             spec_v, en_v, tab_v, res_v):
    wid = lax.axis_index("s") * NC + lax.axis_index("c")
    base = wid * R

    pltpu.sync_copy(species_hbm.at[pl.ds(base * A, R * A)], spec_v)
    pltpu.sync_copy(energies_hbm.at[pl.ds(base, R)], en_v)
    pltpu.sync_copy(table_hbm, tab_v.at[pl.ds(0, 7)])

    lane = lax.iota(jnp.int32, L)

    def row_group(g, _):
        row_base = (g * L + lane) * A

        def col_step(j, acc):
            sv = plsc.load_gather(spec_v, [row_base + j])
            sae = plsc.load_gather(tab_v, [sv])
            return acc + sae

        acc = lax.fori_loop(0, A, col_step, jnp.zeros((L,), jnp.float32),
                            unroll=8)
        res_v[pl.ds(g * L, L)] = acc + en_v[pl.ds(g * L, L)]
        return 0

    lax.fori_loop(0, R // L, row_group, 0)
    pltpu.sync_copy(res_v, out_hbm.at[pl.ds(base, R)])


@jax.jit
def _shift(species, energies, self_energies):
    mesh = plsc.VectorSubcoreMesh(core_axis_name="c", subcore_axis_name="s")
    fn = pl.kernel(
        _sc_body,
        mesh=mesh,
        compiler_params=pltpu.CompilerParams(use_tc_tiling_on_sc=False,
                                             needs_layout_passes=False),
        out_type=jax.ShapeDtypeStruct((B,), jnp.float32),
        scratch_types=[
            pltpu.VMEM((R * A,), jnp.int32),
            pltpu.VMEM((R,), jnp.float32),
            pltpu.VMEM((L,), jnp.float32),
            pltpu.VMEM((R,), jnp.float32),
        ],
    )
    return fn(species, energies, self_energies)


def kernel(species, energies, self_energies):
    out = _shift(species.astype(jnp.int32).reshape(B * A), energies,
                 self_energies)
    return (species, out)


# speedup vs baseline: 1.0460x; 1.0460x over previous
import functools

import jax
import jax.numpy as jnp
from jax import lax
from jax.experimental import pallas as pl
from jax.experimental.pallas import tpu as pltpu
from jax.experimental.pallas import tpu_sc as plsc

B = 16384
A = 200
NC = 2
NS = 16
NW = NC * NS
R = B // NW
L = 16


def _sc_body(species_hbm, energies_hbm, table_hbm, out_hbm,
             spec_v, en_v, tab_v, res_v):
    wid = lax.axis_index("s") * NC + lax.axis_index("c")
    base = wid * R

    pltpu.sync_copy(species_hbm.at[pl.ds(base * A, R * A)], spec_v)
    pltpu.sync_copy(energies_hbm.at[pl.ds(base, R)], en_v)
    pltpu.sync_copy(table_hbm, tab_v.at[pl.ds(0, 7)])

    lane = lax.iota(jnp.int32, L)
    t_vec = tab_v[...]

    def row_group(g, _):
        row_base = (g * L + lane) * A

        def col_step(j, acc):
            sv = plsc.load_gather(spec_v, [row_base + j])
            sae = jnp.take_along_axis(t_vec, sv, axis=0,
                                      mode="promise_in_bounds")
            return acc + sae

        acc = lax.fori_loop(0, A, col_step, jnp.zeros((L,), jnp.float32),
                            unroll=8)
        res_v[pl.ds(g * L, L)] = acc + en_v[pl.ds(g * L, L)]
        return 0

    lax.fori_loop(0, R // L, row_group, 0)
    pltpu.sync_copy(res_v, out_hbm.at[pl.ds(base, R)])


@jax.jit
def _shift(species, energies, self_energies):
    mesh = plsc.VectorSubcoreMesh(core_axis_name="c", subcore_axis_name="s")
    fn = pl.kernel(
        _sc_body,
        mesh=mesh,
        compiler_params=pltpu.CompilerParams(use_tc_tiling_on_sc=False,
                                             needs_layout_passes=False),
        out_type=jax.ShapeDtypeStruct((B,), jnp.float32),
        scratch_types=[
            pltpu.VMEM((R * A,), jnp.int32),
            pltpu.VMEM((R,), jnp.float32),
            pltpu.VMEM((L,), jnp.float32),
            pltpu.VMEM((R,), jnp.float32),
        ],
    )
    return fn(species, energies, self_energies)


def kernel(species, energies, self_energies):
    out = _shift(species.astype(jnp.int32).reshape(B * A), energies,
                 self_energies)
    return (species, out)
